# baseline (speedup 1.0000x reference)
import functools

import jax
import jax.numpy as jnp
from jax.experimental import pallas as pl
from jax.experimental.pallas import tpu as pltpu
from jax.experimental.pallas import tpu_sc as plsc

N, K, D = 18432, 1024, 64
BLK = 512


def _vq_tc_kernel(e2_ref, zt_ref, zz_ref, ee_ref, idx_ref, loss_ref):
    i = pl.program_id(0)
    e2 = e2_ref[...]
    zb = zt_ref[...]
    ze2 = jax.lax.dot_general(e2, zb, (((1,), (1,)), ((), ())),
                              preferred_element_type=jnp.float32)
    d2 = (zz_ref[...] + ze2) + ee_ref[...]
    c = jnp.maximum(d2, 0.0)
    m2 = c[:8]
    for r in range(8, K, 8):
        m2 = jnp.minimum(m2, c[r:r + 8])
    for h in (4, 2, 1):
        m2 = jnp.minimum(m2[:h], m2[h:])
    s = jnp.where(m2 == 0.0, 0.0, m2 * jax.lax.rsqrt(m2))
    iota8 = jax.lax.broadcasted_iota(
        jnp.int32, (8, BLK), 0).astype(jnp.float32)
    ix = jnp.full((8, BLK), float(K), jnp.float32)
    for r in range(0, K, 8):
        cr = c[r:r + 8]
        dr = jnp.where(cr == 0.0, 0.0, cr * jax.lax.rsqrt(cr))
        ix = jnp.minimum(ix, jnp.where(dr == s, iota8 + float(r), float(K)))
    for h in (4, 2, 1):
        ix = jnp.minimum(ix[:h], ix[h:])
    idx_ref[...] = ix.astype(jnp.int32).reshape(1, 1, BLK)
    m2sum = jnp.sum(m2).reshape(1, 1)

    @pl.when(i == 0)
    def _():
        loss_ref[...] = m2sum

    @pl.when(i > 0)
    def _():
        loss_ref[...] += m2sum


def _argmin_distances(zb, e2, zz, ee):
    rows = zb.shape[0]
    grid = rows // BLK
    return pl.pallas_call(
        _vq_tc_kernel,
        grid=(grid,),
        in_specs=[
            pl.BlockSpec((K, D), lambda i: (0, 0)),
            pl.BlockSpec((BLK, D), lambda i: (i, 0)),
            pl.BlockSpec((1, BLK), lambda i: (0, i)),
            pl.BlockSpec((K, 1), lambda i: (0, 0)),
        ],
        out_specs=[
            pl.BlockSpec((1, 1, BLK), lambda i: (i, 0, 0)),
            pl.BlockSpec((1, 1), lambda i: (0, 0)),
        ],
        out_shape=[
            jax.ShapeDtypeStruct((grid, 1, BLK), jnp.int32),
            jax.ShapeDtypeStruct((1, 1), jnp.float32),
        ],
    )(e2, zb, zz, ee)


def _gather_codebook(epad, indices):
    num_indices = indices.shape[0]
    w = 128
    mesh = plsc.VectorSubcoreMesh(core_axis_name="core",
                                  subcore_axis_name="subcore")
    idx2 = indices.reshape(1, num_indices)

    @functools.partial(
        pl.kernel,
        out_type=jax.ShapeDtypeStruct((num_indices, 2 * D), epad.dtype),
        mesh=mesh)
    def k(e_hbm, i_hbm, o_hbm):
        def body(i_vmem, o_vmem):
            pltpu.sync_copy(e_hbm.at[i_vmem.at[0]], o_vmem)

        pltpu.emit_pipeline(
            body,
            grid=(num_indices // w,),
            in_specs=[pl.BlockSpec((1, w), index_map=lambda i: (0, i))],
            out_specs=[pl.BlockSpec((w, 2 * D), index_map=lambda i: (i, 0))],
            core_axis_name=("core", "subcore"),
            dimension_semantics=(pltpu.PARALLEL,),
        )(i_hbm, o_hbm)

    return k(epad, idx2)


def kernel(z, e):
    zz = jnp.sum(z * z, axis=1)[None, :]
    ee = jnp.sum(e * e, axis=1, keepdims=True)
    e2 = -2.0 * e
    epad = jnp.concatenate([e, jnp.zeros((K, D), e.dtype)], axis=1)
    idx3, loss_sum = _argmin_distances(z, e2, zz, ee)
    min_indices = idx3.reshape(N)
    zq = _gather_codebook(epad, min_indices)[:, :D]
    commit_loss = loss_sum[0, 0] / (N * D)
    return zq, min_indices, commit_loss


# speedup vs baseline: 1.1021x; 1.1021x over previous
import functools

import jax
import jax.numpy as jnp
from jax.experimental import pallas as pl
from jax.experimental.pallas import tpu as pltpu
from jax.experimental.pallas import tpu_sc as plsc

N, K, D = 18432, 1024, 64
BLK = 1024


def _vq_tc_kernel(e2_ref, zt_ref, zz_ref, ee_ref, idx_ref, loss_ref):
    i = pl.program_id(0)
    e2 = e2_ref[...]
    zt = zt_ref[...]
    ze2 = jax.lax.dot_general(e2, zt, (((1,), (0,)), ((), ())),
                              preferred_element_type=jnp.float32)
    d2 = (zz_ref[...] + ze2) + ee_ref[...]
    c = jnp.maximum(d2, 0.0)
    m2 = c[:8]
    for r in range(8, K, 8):
        m2 = jnp.minimum(m2, c[r:r + 8])
    for h in (4, 2, 1):
        m2 = jnp.minimum(m2[:h], m2[h:])
    s = jnp.where(m2 == 0.0, 0.0, m2 * jax.lax.rsqrt(m2))
    iota8 = jax.lax.broadcasted_iota(
        jnp.int32, (8, BLK), 0).astype(jnp.float32)
    ix = jnp.full((8, BLK), float(K), jnp.float32)
    for r in range(0, K, 8):
        cr = c[r:r + 8]
        dr = jnp.where(cr == 0.0, 0.0, cr * jax.lax.rsqrt(cr))
        ix = jnp.minimum(ix, jnp.where(dr == s, iota8 + float(r), float(K)))
    for h in (4, 2, 1):
        ix = jnp.minimum(ix[:h], ix[h:])
    idx_ref[...] = ix.astype(jnp.int32).reshape(1, 1, BLK)
    m2sum = jnp.sum(m2).reshape(1, 1)

    @pl.when(i == 0)
    def _():
        loss_ref[...] = m2sum

    @pl.when(i > 0)
    def _():
        loss_ref[...] += m2sum


def _argmin_distances(zt, e2, zz, ee):
    rows = zt.shape[1]
    grid = rows // BLK
    return pl.pallas_call(
        _vq_tc_kernel,
        grid=(grid,),
        in_specs=[
            pl.BlockSpec((K, D), lambda i: (0, 0)),
            pl.BlockSpec((D, BLK), lambda i: (0, i)),
            pl.BlockSpec((1, BLK), lambda i: (0, i)),
            pl.BlockSpec((K, 1), lambda i: (0, 0)),
        ],
        out_specs=[
            pl.BlockSpec((1, 1, BLK), lambda i: (i, 0, 0)),
            pl.BlockSpec((1, 1), lambda i: (0, 0)),
        ],
        out_shape=[
            jax.ShapeDtypeStruct((grid, 1, BLK), jnp.int32),
            jax.ShapeDtypeStruct((1, 1), jnp.float32),
        ],
    )(e2, zt, zz, ee)


def _gather_codebook(epad, indices):
    num_indices = indices.shape[0]
    w = 128
    mesh = plsc.VectorSubcoreMesh(core_axis_name="core",
                                  subcore_axis_name="subcore")
    idx2 = indices.reshape(1, num_indices)

    @functools.partial(
        pl.kernel,
        out_type=jax.ShapeDtypeStruct((num_indices, 2 * D), epad.dtype),
        mesh=mesh)
    def k(e_hbm, i_hbm, o_hbm):
        def body(i_vmem, o_vmem):
            pltpu.sync_copy(e_hbm.at[i_vmem.at[0]], o_vmem)

        pltpu.emit_pipeline(
            body,
            grid=(num_indices // w,),
            in_specs=[pl.BlockSpec((1, w), index_map=lambda i: (0, i))],
            out_specs=[pl.BlockSpec((w, 2 * D), index_map=lambda i: (i, 0))],
            core_axis_name=("core", "subcore"),
            dimension_semantics=(pltpu.PARALLEL,),
        )(i_hbm, o_hbm)

    return k(epad, idx2)


def kernel(z, e):
    zz = jnp.sum(z * z, axis=1)[None, :]
    ee = jnp.sum(e * e, axis=1, keepdims=True)
    zt = z.T
    e2 = -2.0 * e
    epad = jnp.concatenate([e, jnp.zeros((K, D), e.dtype)], axis=1)
    idx3, loss_sum = _argmin_distances(zt, e2, zz, ee)
    min_indices = idx3.reshape(N)
    zq = _gather_codebook(epad, min_indices)[:, :D]
    commit_loss = loss_sum[0, 0] / (N * D)
    return zq, min_indices, commit_loss


# speedup vs baseline: 1.1206x; 1.0168x over previous
import functools

import jax
import jax.numpy as jnp
from jax.experimental import pallas as pl
from jax.experimental.pallas import tpu as pltpu
from jax.experimental.pallas import tpu_sc as plsc

N, K, D = 18432, 1024, 64
BLK = 2048


def _vq_tc_kernel(e2_ref, zt_ref, zz_ref, ee_ref, idx_ref, loss_ref):
    i = pl.program_id(0)
    e2 = e2_ref[...]
    zt = zt_ref[...]
    ze2 = jax.lax.dot_general(e2, zt, (((1,), (0,)), ((), ())),
                              preferred_element_type=jnp.float32)
    d2 = (zz_ref[...] + ze2) + ee_ref[...]
    c = jnp.maximum(d2, 0.0)
    m2 = c[:8]
    for r in range(8, K, 8):
        m2 = jnp.minimum(m2, c[r:r + 8])
    for h in (4, 2, 1):
        m2 = jnp.minimum(m2[:h], m2[h:])
    s = jnp.where(m2 == 0.0, 0.0, m2 * jax.lax.rsqrt(m2))
    iota8 = jax.lax.broadcasted_iota(
        jnp.int32, (8, BLK), 0).astype(jnp.float32)
    ix = jnp.full((8, BLK), float(K), jnp.float32)
    for r in range(0, K, 8):
        cr = c[r:r + 8]
        dr = jnp.where(cr == 0.0, 0.0, cr * jax.lax.rsqrt(cr))
        ix = jnp.minimum(ix, jnp.where(dr == s, iota8 + float(r), float(K)))
    for h in (4, 2, 1):
        ix = jnp.minimum(ix[:h], ix[h:])
    idx_ref[...] = ix.astype(jnp.int32).reshape(1, 1, BLK)
    m2sum = jnp.sum(m2).reshape(1, 1)

    @pl.when(i == 0)
    def _():
        loss_ref[...] = m2sum

    @pl.when(i > 0)
    def _():
        loss_ref[...] += m2sum


def _argmin_distances(zt, e2, zz, ee):
    rows = zt.shape[1]
    grid = rows // BLK
    return pl.pallas_call(
        _vq_tc_kernel,
        grid=(grid,),
        in_specs=[
            pl.BlockSpec((K, D), lambda i: (0, 0)),
            pl.BlockSpec((D, BLK), lambda i: (0, i)),
            pl.BlockSpec((1, BLK), lambda i: (0, i)),
            pl.BlockSpec((K, 1), lambda i: (0, 0)),
        ],
        out_specs=[
            pl.BlockSpec((1, 1, BLK), lambda i: (i, 0, 0)),
            pl.BlockSpec((1, 1), lambda i: (0, 0)),
        ],
        out_shape=[
            jax.ShapeDtypeStruct((grid, 1, BLK), jnp.int32),
            jax.ShapeDtypeStruct((1, 1), jnp.float32),
        ],
    )(e2, zt, zz, ee)


def _gather_codebook(epad, indices):
    num_indices = indices.shape[0]
    w = 128
    mesh = plsc.VectorSubcoreMesh(core_axis_name="core",
                                  subcore_axis_name="subcore")
    idx2 = indices.reshape(1, num_indices)

    @functools.partial(
        pl.kernel,
        out_type=jax.ShapeDtypeStruct((num_indices, 2 * D), epad.dtype),
        mesh=mesh)
    def k(e_hbm, i_hbm, o_hbm):
        def body(i_vmem, o_vmem):
            pltpu.sync_copy(e_hbm.at[i_vmem.at[0]], o_vmem)

        pltpu.emit_pipeline(
            body,
            grid=(num_indices // w,),
            in_specs=[pl.BlockSpec((1, w), index_map=lambda i: (0, i))],
            out_specs=[pl.BlockSpec((w, 2 * D), index_map=lambda i: (i, 0))],
            core_axis_name=("core", "subcore"),
            dimension_semantics=(pltpu.PARALLEL,),
        )(i_hbm, o_hbm)

    return k(epad, idx2)


def kernel(z, e):
    zz = jnp.sum(z * z, axis=1)[None, :]
    ee = jnp.sum(e * e, axis=1, keepdims=True)
    zt = z.T
    e2 = -2.0 * e
    epad = jnp.concatenate([e, jnp.zeros((K, D), e.dtype)], axis=1)
    idx3, loss_sum = _argmin_distances(zt, e2, zz, ee)
    min_indices = idx3.reshape(N)
    zq = _gather_codebook(epad, min_indices)[:, :D]
    commit_loss = loss_sum[0, 0] / (N * D)
    return zq, min_indices, commit_loss


# speedup vs baseline: 1.1460x; 1.0226x over previous
import functools

import jax
import jax.numpy as jnp
from jax.experimental import pallas as pl
from jax.experimental.pallas import tpu as pltpu
from jax.experimental.pallas import tpu_sc as plsc

N, K, D = 18432, 1024, 64
BLK = 3072


def _vq_tc_kernel(e2_ref, zt_ref, zz_ref, ee_ref, idx_ref, loss_ref):
    i = pl.program_id(0)
    e2 = e2_ref[...]
    zt = zt_ref[...]
    ze2 = jax.lax.dot_general(e2, zt, (((1,), (0,)), ((), ())),
                              preferred_element_type=jnp.float32)
    d2 = (zz_ref[...] + ze2) + ee_ref[...]
    c = jnp.maximum(d2, 0.0)
    m2 = c[:8]
    for r in range(8, K, 8):
        m2 = jnp.minimum(m2, c[r:r + 8])
    for h in (4, 2, 1):
        m2 = jnp.minimum(m2[:h], m2[h:])
    s = jnp.where(m2 == 0.0, 0.0, m2 * jax.lax.rsqrt(m2))
    iota8 = jax.lax.broadcasted_iota(
        jnp.int32, (8, BLK), 0).astype(jnp.float32)
    ix = jnp.full((8, BLK), float(K), jnp.float32)
    for r in range(0, K, 8):
        cr = c[r:r + 8]
        dr = jnp.where(cr == 0.0, 0.0, cr * jax.lax.rsqrt(cr))
        ix = jnp.minimum(ix, jnp.where(dr == s, iota8 + float(r), float(K)))
    for h in (4, 2, 1):
        ix = jnp.minimum(ix[:h], ix[h:])
    idx_ref[...] = ix.astype(jnp.int32).reshape(1, 1, BLK)
    m2sum = jnp.sum(m2).reshape(1, 1)

    @pl.when(i == 0)
    def _():
        loss_ref[...] = m2sum

    @pl.when(i > 0)
    def _():
        loss_ref[...] += m2sum


def _argmin_distances(zt, e2, zz, ee):
    rows = zt.shape[1]
    grid = rows // BLK
    return pl.pallas_call(
        _vq_tc_kernel,
        grid=(grid,),
        in_specs=[
            pl.BlockSpec((K, D), lambda i: (0, 0)),
            pl.BlockSpec((D, BLK), lambda i: (0, i)),
            pl.BlockSpec((1, BLK), lambda i: (0, i)),
            pl.BlockSpec((K, 1), lambda i: (0, 0)),
        ],
        out_specs=[
            pl.BlockSpec((1, 1, BLK), lambda i: (i, 0, 0)),
            pl.BlockSpec((1, 1), lambda i: (0, 0)),
        ],
        out_shape=[
            jax.ShapeDtypeStruct((grid, 1, BLK), jnp.int32),
            jax.ShapeDtypeStruct((1, 1), jnp.float32),
        ],
    )(e2, zt, zz, ee)


def _gather_codebook(epad, indices):
    num_indices = indices.shape[0]
    w = 128
    mesh = plsc.VectorSubcoreMesh(core_axis_name="core",
                                  subcore_axis_name="subcore")
    idx2 = indices.reshape(1, num_indices)

    @functools.partial(
        pl.kernel,
        out_type=jax.ShapeDtypeStruct((num_indices, 2 * D), epad.dtype),
        mesh=mesh)
    def k(e_hbm, i_hbm, o_hbm):
        def body(i_vmem, o_vmem):
            pltpu.sync_copy(e_hbm.at[i_vmem.at[0]], o_vmem)

        pltpu.emit_pipeline(
            body,
            grid=(num_indices // w,),
            in_specs=[pl.BlockSpec((1, w), index_map=lambda i: (0, i))],
            out_specs=[pl.BlockSpec((w, 2 * D), index_map=lambda i: (i, 0))],
            core_axis_name=("core", "subcore"),
            dimension_semantics=(pltpu.PARALLEL,),
        )(i_hbm, o_hbm)

    return k(epad, idx2)


def kernel(z, e):
    zz = jnp.sum(z * z, axis=1)[None, :]
    ee = jnp.sum(e * e, axis=1, keepdims=True)
    zt = z.T
    e2 = -2.0 * e
    epad = jnp.concatenate([e, jnp.zeros((K, D), e.dtype)], axis=1)
    idx3, loss_sum = _argmin_distances(zt, e2, zz, ee)
    min_indices = idx3.reshape(N)
    zq = _gather_codebook(epad, min_indices)[:, :D]
    commit_loss = loss_sum[0, 0] / (N * D)
    return zq, min_indices, commit_loss


# speedup vs baseline: 1.1512x; 1.0045x over previous
import functools

import jax
import jax.numpy as jnp
from jax.experimental import pallas as pl
from jax.experimental.pallas import tpu as pltpu
from jax.experimental.pallas import tpu_sc as plsc

N, K, D = 18432, 1024, 64
BLK = 3072


def _vq_tc_kernel(e2_ref, zt_ref, zz_ref, ee_ref, idx_ref, loss_ref):
    i = pl.program_id(0)
    e2 = e2_ref[...]
    zt = zt_ref[...]
    ze2 = jax.lax.dot_general(e2, zt, (((1,), (0,)), ((), ())),
                              preferred_element_type=jnp.float32)
    d2 = (zz_ref[...] + ze2) + ee_ref[...]
    c = jnp.maximum(d2, 0.0)
    m2 = c[:8]
    for r in range(8, K, 8):
        m2 = jnp.minimum(m2, c[r:r + 8])
    for h in (4, 2, 1):
        m2 = jnp.minimum(m2[:h], m2[h:])
    s = jnp.where(m2 == 0.0, 0.0, m2 * jax.lax.rsqrt(m2))
    iota8 = jax.lax.broadcasted_iota(
        jnp.int32, (8, BLK), 0).astype(jnp.float32)
    ix = jnp.full((8, BLK), float(K), jnp.float32)
    for r in range(0, K, 8):
        cr = c[r:r + 8]
        dr = jnp.where(cr == 0.0, 0.0, cr * jax.lax.rsqrt(cr))
        ix = jnp.minimum(ix, jnp.where(dr == s, iota8 + float(r), float(K)))
    for h in (4, 2, 1):
        ix = jnp.minimum(ix[:h], ix[h:])
    idx_ref[...] = ix.astype(jnp.int32).reshape(1, 1, BLK)
    m2sum = jnp.sum(m2).reshape(1, 1)

    @pl.when(i == 0)
    def _():
        loss_ref[...] = m2sum

    @pl.when(i > 0)
    def _():
        loss_ref[...] += m2sum


def _argmin_distances(zt, e2, zz, ee):
    rows = zt.shape[1]
    grid = rows // BLK
    return pl.pallas_call(
        _vq_tc_kernel,
        grid=(grid,),
        in_specs=[
            pl.BlockSpec((K, D), lambda i: (0, 0)),
            pl.BlockSpec((D, BLK), lambda i: (0, i)),
            pl.BlockSpec((1, BLK), lambda i: (0, i)),
            pl.BlockSpec((K, 1), lambda i: (0, 0)),
        ],
        out_specs=[
            pl.BlockSpec((1, 1, BLK), lambda i: (i, 0, 0)),
            pl.BlockSpec((1, 1), lambda i: (0, 0)),
        ],
        out_shape=[
            jax.ShapeDtypeStruct((grid, 1, BLK), jnp.int32),
            jax.ShapeDtypeStruct((1, 1), jnp.float32),
        ],
    )(e2, zt, zz, ee)


def _gather_codebook(epad, indices):
    num_indices = indices.shape[0]
    w = 256
    mesh = plsc.VectorSubcoreMesh(core_axis_name="core",
                                  subcore_axis_name="subcore")
    idx2 = indices.reshape(1, num_indices)

    @functools.partial(
        pl.kernel,
        out_type=jax.ShapeDtypeStruct((num_indices, 2 * D), epad.dtype),
        mesh=mesh)
    def k(e_hbm, i_hbm, o_hbm):
        def body(i_vmem, o_vmem):
            pltpu.sync_copy(e_hbm.at[i_vmem.at[0]], o_vmem)

        pltpu.emit_pipeline(
            body,
            grid=(num_indices // w,),
            in_specs=[pl.BlockSpec((1, w), index_map=lambda i: (0, i))],
            out_specs=[pl.BlockSpec((w, 2 * D), index_map=lambda i: (i, 0))],
            core_axis_name=("core", "subcore"),
            dimension_semantics=(pltpu.PARALLEL,),
        )(i_hbm, o_hbm)

    return k(epad, idx2)


def kernel(z, e):
    zz = jnp.sum(z * z, axis=1)[None, :]
    ee = jnp.sum(e * e, axis=1, keepdims=True)
    zt = z.T
    e2 = -2.0 * e
    epad = jnp.concatenate([e, jnp.zeros((K, D), e.dtype)], axis=1)
    idx3, loss_sum = _argmin_distances(zt, e2, zz, ee)
    min_indices = idx3.reshape(N)
    zq = _gather_codebook(epad, min_indices)[:, :D]
    commit_loss = loss_sum[0, 0] / (N * D)
    return zq, min_indices, commit_loss
